# baseline (speedup 1.0000x reference)
import functools

import jax
import jax.numpy as jnp
from jax.experimental import pallas as pl
from jax.experimental.pallas import tpu as pltpu

EMBED_DIM = 768
FFN_DIM = 3072
NUM_HEADS = 12
HEAD_DIM = EMBED_DIM // NUM_HEADS
CHUNK_LEN = 256
REG_LEN = 2048
NUM_CHUNKS = REG_LEN // CHUNK_LEN
SUM_LEN = NUM_CHUNKS
NSUM = NUM_HEADS * SUM_LEN
SCALE = 1.0 / (HEAD_DIM ** 0.5)


def _ln(x, g, b):
    m = jnp.mean(x, axis=-1, keepdims=True)
    v = jnp.mean((x - m) ** 2, axis=-1, keepdims=True)
    return (x - m) * jax.lax.rsqrt(v + 1e-5) * g + b


def _dot(a, b):
    return jnp.dot(a, b, preferred_element_type=jnp.float32)


def _dot_t(a, b):
    return jax.lax.dot_general(a, b, (((1,), (1,)), ((), ())),
                               preferred_element_type=jnp.float32)


def _dot_tl(a, b):
    return jax.lax.dot_general(a, b, (((0,), (0,)), ((), ())),
                               preferred_element_type=jnp.float32)


def _body(reg_x_ref, sum_x_ref, wq_ref, wk_ref, wv_ref, wo_ref,
          reg_ln_g_ref, reg_ln_b_ref, sum_ln_g_ref, sum_ln_b_ref,
          reg_fln_g_ref, reg_fln_b_ref, sum_fln_g_ref, sum_fln_b_ref,
          rfc1w_ref, rfc1b_ref, rfc2w_ref, rfc2b_ref,
          sfc1w_ref, sfc1b_ref, sfc2w_ref, sfc2b_ref,
          out_reg_ref, out_sum_ref,
          qs_ref, ks_ref, vs_ref, kbd_ref, qbd_ref, vbd_ref,
          ssc_ref, vall_ref):
    c = pl.program_id(0)

    @pl.when(c == 0)
    def _init_summary_qkv():
        hs = _ln(sum_x_ref[...], sum_ln_g_ref[...], sum_ln_b_ref[...])
        q_sum = _dot(hs, wq_ref[...]) * SCALE
        k_sum = _dot(hs, wk_ref[...])
        v_sum = _dot(hs, wv_ref[...])
        qs_ref[...] = q_sum
        ks_ref[...] = k_sum
        vs_ref[...] = v_sum
        kbd_ref[...] = jnp.zeros((EMBED_DIM, NSUM), jnp.float32)
        qbd_ref[...] = jnp.zeros((EMBED_DIM, NSUM), jnp.float32)
        vbd_ref[...] = jnp.zeros((NSUM, EMBED_DIM), jnp.float32)
        for hd in range(NUM_HEADS):
            sl = slice(hd * HEAD_DIM, (hd + 1) * HEAD_DIM)
            ssl = slice(hd * SUM_LEN, (hd + 1) * SUM_LEN)
            kbd_ref[sl, ssl] = k_sum[:, sl].T
            qbd_ref[sl, ssl] = q_sum[:, sl].T
            vbd_ref[ssl, sl] = v_sum[:, sl]

    x0 = reg_x_ref[...]
    h = _ln(x0, reg_ln_g_ref[...], reg_ln_b_ref[...])
    q = _dot(h, wq_ref[...]) * SCALE
    k = _dot(h, wk_ref[...])
    v = _dot(h, wv_ref[...])
    vall_ref[pl.ds(c * CHUNK_LEN, CHUNK_LEN), :] = v

    colbd = jax.lax.broadcasted_iota(jnp.int32, (CHUNK_LEN, NSUM), 1)
    sum_key_f = ((colbd & (SUM_LEN - 1)) < c).astype(jnp.float32)
    e_sm_all = jnp.exp(_dot(q, kbd_ref[...])) * sum_key_f
    ctx_sum_all = _dot(e_sm_all, vbd_ref[...])
    rowo = jax.lax.broadcasted_iota(jnp.int32, (NSUM, NUM_HEADS), 0)
    colo = jax.lax.broadcasted_iota(jnp.int32, (NSUM, NUM_HEADS), 1)
    ones_bd = ((rowo // SUM_LEN) == colo).astype(jnp.float32)
    l_sm12 = _dot(e_sm_all, ones_bd)

    ssc_ref[pl.ds(c * CHUNK_LEN, CHUNK_LEN), :] = _dot(k, qbd_ref[...])

    row = jax.lax.broadcasted_iota(jnp.int32, (CHUNK_LEN, CHUNK_LEN), 0)
    col = jax.lax.broadcasted_iota(jnp.int32, (CHUNK_LEN, CHUNK_LEN), 1)
    causal_f = (row >= col).astype(jnp.float32)

    ctxs = []
    for hd in range(NUM_HEADS):
        sl = slice(hd * HEAD_DIM, (hd + 1) * HEAD_DIM)
        qh, kh, vh = q[:, sl], k[:, sl], v[:, sl]
        e_loc = jnp.exp(_dot_t(qh, kh)) * causal_f
        l = (jnp.sum(e_loc, axis=-1, keepdims=True)
             + l_sm12[:, hd:hd + 1])
        ctxs.append((_dot(e_loc, vh) + ctx_sum_all[:, sl]) * (1.0 / l))

    ctx = jnp.concatenate(ctxs, axis=1)
    x = x0 + _dot(ctx, wo_ref[...])
    f = _ln(x, reg_fln_g_ref[...], reg_fln_b_ref[...])
    ffn = jnp.maximum(_dot(f, rfc1w_ref[...]) + rfc1b_ref[...], 0.0)
    out_reg_ref[...] = x + _dot(ffn, rfc2w_ref[...]) + rfc2b_ref[...]

    @pl.when(c == NUM_CHUNKS - 1)
    def _finalize_summary():
        q_sum = qs_ref[...]
        k_sum = ks_ref[...]
        v_sum = vs_ref[...]
        rowr = jax.lax.broadcasted_iota(jnp.int32, (REG_LEN, NSUM), 0)
        colr = jax.lax.broadcasted_iota(jnp.int32, (REG_LEN, NSUM), 1)
        sr_f = ((rowr // CHUNK_LEN) <= (colr & (SUM_LEN - 1))).astype(
            jnp.float32)
        e_sr_all = jnp.exp(ssc_ref[...]) * sr_f
        ctx_sr = _dot_tl(e_sr_all, vall_ref[...])
        l_sr = jnp.sum(e_sr_all, axis=0, keepdims=True)
        row8 = jax.lax.broadcasted_iota(jnp.int32, (SUM_LEN, SUM_LEN), 0)
        col8 = jax.lax.broadcasted_iota(jnp.int32, (SUM_LEN, SUM_LEN), 1)
        ss_f = (col8 <= row8).astype(jnp.float32)
        ctxs_s = []
        for hd in range(NUM_HEADS):
            sl = slice(hd * HEAD_DIM, (hd + 1) * HEAD_DIM)
            ssl = slice(hd * SUM_LEN, (hd + 1) * SUM_LEN)
            e_ss = jnp.exp(_dot_t(q_sum[:, sl], k_sum[:, sl])) * ss_f
            l = (jnp.sum(e_ss, axis=-1, keepdims=True)
                 + l_sr[:, ssl].T)
            ctxs_s.append((_dot(e_ss, v_sum[:, sl]) + ctx_sr[ssl, sl])
                          * (1.0 / l))
        ctx_s = jnp.concatenate(ctxs_s, axis=1)
        xs = sum_x_ref[...] + _dot(ctx_s, wo_ref[...])
        fs = _ln(xs, sum_fln_g_ref[...], sum_fln_b_ref[...])
        ffn_s = jnp.maximum(_dot(fs, sfc1w_ref[...]) + sfc1b_ref[...], 0.0)
        out_sum_ref[...] = xs + _dot(ffn_s, sfc2w_ref[...]) + sfc2b_ref[...]


@functools.partial(jax.jit, static_argnames=("interpret",))
def _run(reg_x, sum_x, Wq, Wk, Wv, Wo, reg_ln_g, reg_ln_b, sum_ln_g, sum_ln_b,
         reg_fln_g, reg_fln_b, sum_fln_g, sum_fln_b,
         reg_fc1_w, reg_fc1_b, reg_fc2_w, reg_fc2_b,
         sum_fc1_w, sum_fc1_b, sum_fc2_w, sum_fc2_b, interpret=False):
    full = lambda shape: pl.BlockSpec(shape, lambda c: (0,) * len(shape))
    in_specs = [
        pl.BlockSpec((CHUNK_LEN, EMBED_DIM), lambda c: (c, 0)),
        full((SUM_LEN, EMBED_DIM)),
        full((EMBED_DIM, EMBED_DIM)),
        full((EMBED_DIM, EMBED_DIM)),
        full((EMBED_DIM, EMBED_DIM)),
        full((EMBED_DIM, EMBED_DIM)),
        full((1, EMBED_DIM)), full((1, EMBED_DIM)),
        full((1, EMBED_DIM)), full((1, EMBED_DIM)),
        full((1, EMBED_DIM)), full((1, EMBED_DIM)),
        full((1, EMBED_DIM)), full((1, EMBED_DIM)),
        full((EMBED_DIM, FFN_DIM)), full((1, FFN_DIM)),
        full((FFN_DIM, EMBED_DIM)), full((1, EMBED_DIM)),
        full((EMBED_DIM, FFN_DIM)), full((1, FFN_DIM)),
        full((FFN_DIM, EMBED_DIM)), full((1, EMBED_DIM)),
    ]
    out_specs = [
        pl.BlockSpec((CHUNK_LEN, EMBED_DIM), lambda c: (c, 0)),
        full((SUM_LEN, EMBED_DIM)),
    ]
    out_reg, out_sum = pl.pallas_call(
        _body,
        grid=(NUM_CHUNKS,),
        in_specs=in_specs,
        out_specs=out_specs,
        out_shape=[
            jax.ShapeDtypeStruct((REG_LEN, EMBED_DIM), jnp.float32),
            jax.ShapeDtypeStruct((SUM_LEN, EMBED_DIM), jnp.float32),
        ],
        scratch_shapes=[
            pltpu.VMEM((SUM_LEN, EMBED_DIM), jnp.float32),
            pltpu.VMEM((SUM_LEN, EMBED_DIM), jnp.float32),
            pltpu.VMEM((SUM_LEN, EMBED_DIM), jnp.float32),
            pltpu.VMEM((EMBED_DIM, NSUM), jnp.float32),
            pltpu.VMEM((EMBED_DIM, NSUM), jnp.float32),
            pltpu.VMEM((NSUM, EMBED_DIM), jnp.float32),
            pltpu.VMEM((REG_LEN, NSUM), jnp.float32),
            pltpu.VMEM((REG_LEN, EMBED_DIM), jnp.float32),
        ],
        compiler_params=pltpu.CompilerParams(
            vmem_limit_bytes=63 * 1024 * 1024),
        interpret=interpret,
    )(
        reg_x[0], sum_x[0], Wq, Wk, Wv, Wo,
        reg_ln_g[None], reg_ln_b[None], sum_ln_g[None], sum_ln_b[None],
        reg_fln_g[None], reg_fln_b[None], sum_fln_g[None], sum_fln_b[None],
        reg_fc1_w, reg_fc1_b[None], reg_fc2_w, reg_fc2_b[None],
        sum_fc1_w, sum_fc1_b[None], sum_fc2_w, sum_fc2_b[None],
    )
    return jnp.concatenate([out_sum, out_reg], axis=0)[None]


def kernel(reg_x, sum_x, Wq, Wk, Wv, Wo, reg_ln_g, reg_ln_b, sum_ln_g,
           sum_ln_b, reg_fln_g, reg_fln_b, sum_fln_g, sum_fln_b,
           reg_fc1_w, reg_fc1_b, reg_fc2_w, reg_fc2_b,
           sum_fc1_w, sum_fc1_b, sum_fc2_w, sum_fc2_b):
    return _run(reg_x, sum_x, Wq, Wk, Wv, Wo, reg_ln_g, reg_ln_b, sum_ln_g,
                sum_ln_b, reg_fln_g, reg_fln_b, sum_fln_g, sum_fln_b,
                reg_fc1_w, reg_fc1_b, reg_fc2_w, reg_fc2_b,
                sum_fc1_w, sum_fc1_b, sum_fc2_w, sum_fc2_b)


# speedup vs baseline: 1.1905x; 1.1905x over previous
import functools

import jax
import jax.numpy as jnp
from jax.experimental import pallas as pl
from jax.experimental.pallas import tpu as pltpu

EMBED_DIM = 768
FFN_DIM = 3072
NUM_HEADS = 12
HEAD_DIM = EMBED_DIM // NUM_HEADS
CHUNK_LEN = 256
REG_LEN = 2048
NUM_CHUNKS = REG_LEN // CHUNK_LEN
SUM_LEN = NUM_CHUNKS
NSUM = NUM_HEADS * SUM_LEN
SCALE = 1.0 / (HEAD_DIM ** 0.5)


def _ln(x, g, b):
    m = jnp.mean(x, axis=-1, keepdims=True)
    v = jnp.mean((x - m) ** 2, axis=-1, keepdims=True)
    return (x - m) * jax.lax.rsqrt(v + 1e-5) * g + b


def _dot(a, b):
    return jnp.dot(a, b, preferred_element_type=jnp.float32)


def _dot_t(a, b):
    return jax.lax.dot_general(a, b, (((1,), (1,)), ((), ())),
                               preferred_element_type=jnp.float32)


def _dot_tl(a, b):
    return jax.lax.dot_general(a, b, (((0,), (0,)), ((), ())),
                               preferred_element_type=jnp.float32)


def _body(reg_x_ref, sum_x_ref, wq_ref, wk_ref, wv_ref, wo_ref,
          reg_ln_g_ref, reg_ln_b_ref, sum_ln_g_ref, sum_ln_b_ref,
          reg_fln_g_ref, reg_fln_b_ref, sum_fln_g_ref, sum_fln_b_ref,
          rfc1w_ref, rfc1b_ref, rfc2w_ref, rfc2b_ref,
          sfc1w_ref, sfc1b_ref, sfc2w_ref, sfc2b_ref,
          out_reg_ref, out_sum_ref,
          qs_ref, ks_ref, vs_ref, qbd_ref, ssc_ref, vall_ref):
    c = pl.program_id(0)

    @pl.when(c == 0)
    def _init_summary_qkv():
        hs = _ln(sum_x_ref[...], sum_ln_g_ref[...], sum_ln_b_ref[...])
        q_sum = _dot(hs, wq_ref[...]) * SCALE
        k_sum = _dot(hs, wk_ref[...])
        v_sum = _dot(hs, wv_ref[...])
        qs_ref[...] = q_sum
        ks_ref[...] = k_sum
        vs_ref[...] = v_sum
        qbd_ref[...] = jnp.zeros((EMBED_DIM, NSUM), jnp.float32)
        for hd in range(NUM_HEADS):
            sl = slice(hd * HEAD_DIM, (hd + 1) * HEAD_DIM)
            ssl = slice(hd * SUM_LEN, (hd + 1) * SUM_LEN)
            qbd_ref[sl, ssl] = q_sum[:, sl].T

    x0 = reg_x_ref[...]
    h = _ln(x0, reg_ln_g_ref[...], reg_ln_b_ref[...])
    q = _dot(h, wq_ref[...]) * SCALE
    k = _dot(h, wk_ref[...])
    v = _dot(h, wv_ref[...])
    vall_ref[pl.ds(c * CHUNK_LEN, CHUNK_LEN), :] = v

    q_sum = qs_ref[...]
    k_sum = ks_ref[...]
    v_sum = vs_ref[...]

    ssc_ref[pl.ds(c * CHUNK_LEN, CHUNK_LEN), :] = _dot(k, qbd_ref[...])

    row = jax.lax.broadcasted_iota(jnp.int32, (CHUNK_LEN, CHUNK_LEN), 0)
    col = jax.lax.broadcasted_iota(jnp.int32, (CHUNK_LEN, CHUNK_LEN), 1)
    causal_f = (row >= col).astype(jnp.float32)
    col_s = jax.lax.broadcasted_iota(jnp.int32, (CHUNK_LEN, SUM_LEN), 1)
    sum_key_f = (col_s < c).astype(jnp.float32)

    ctxs = []
    for hd in range(NUM_HEADS):
        sl = slice(hd * HEAD_DIM, (hd + 1) * HEAD_DIM)
        qh, kh, vh = q[:, sl], k[:, sl], v[:, sl]
        e_loc = jnp.exp(_dot_t(qh, kh)) * causal_f
        e_sm = jnp.exp(_dot_t(qh, k_sum[:, sl])) * sum_key_f
        l = (jnp.sum(e_loc, axis=-1, keepdims=True)
             + jnp.sum(e_sm, axis=-1, keepdims=True))
        ctxs.append((_dot(e_loc, vh) + _dot(e_sm, v_sum[:, sl])) * (1.0 / l))

    ctx = jnp.concatenate(ctxs, axis=1)
    x = x0 + _dot(ctx, wo_ref[...])
    f = _ln(x, reg_fln_g_ref[...], reg_fln_b_ref[...])
    ffn = jnp.maximum(_dot(f, rfc1w_ref[...]) + rfc1b_ref[...], 0.0)
    out_reg_ref[...] = x + _dot(ffn, rfc2w_ref[...]) + rfc2b_ref[...]

    @pl.when(c == NUM_CHUNKS - 1)
    def _finalize_summary():
        q_sum = qs_ref[...]
        k_sum = ks_ref[...]
        v_sum = vs_ref[...]
        rowr = jax.lax.broadcasted_iota(jnp.int32, (REG_LEN, NSUM), 0)
        colr = jax.lax.broadcasted_iota(jnp.int32, (REG_LEN, NSUM), 1)
        sr_f = ((rowr // CHUNK_LEN) <= (colr & (SUM_LEN - 1))).astype(
            jnp.float32)
        e_sr_all = jnp.exp(ssc_ref[...]) * sr_f
        ctx_sr = _dot_tl(e_sr_all, vall_ref[...])
        l_sr = jnp.sum(e_sr_all, axis=0, keepdims=True)
        row8 = jax.lax.broadcasted_iota(jnp.int32, (SUM_LEN, SUM_LEN), 0)
        col8 = jax.lax.broadcasted_iota(jnp.int32, (SUM_LEN, SUM_LEN), 1)
        ss_f = (col8 <= row8).astype(jnp.float32)
        ctxs_s = []
        for hd in range(NUM_HEADS):
            sl = slice(hd * HEAD_DIM, (hd + 1) * HEAD_DIM)
            ssl = slice(hd * SUM_LEN, (hd + 1) * SUM_LEN)
            e_ss = jnp.exp(_dot_t(q_sum[:, sl], k_sum[:, sl])) * ss_f
            l = (jnp.sum(e_ss, axis=-1, keepdims=True)
                 + l_sr[:, ssl].T)
            ctxs_s.append((_dot(e_ss, v_sum[:, sl]) + ctx_sr[ssl, sl])
                          * (1.0 / l))
        ctx_s = jnp.concatenate(ctxs_s, axis=1)
        xs = sum_x_ref[...] + _dot(ctx_s, wo_ref[...])
        fs = _ln(xs, sum_fln_g_ref[...], sum_fln_b_ref[...])
        ffn_s = jnp.maximum(_dot(fs, sfc1w_ref[...]) + sfc1b_ref[...], 0.0)
        out_sum_ref[...] = xs + _dot(ffn_s, sfc2w_ref[...]) + sfc2b_ref[...]


@functools.partial(jax.jit, static_argnames=("interpret",))
def _run(reg_x, sum_x, Wq, Wk, Wv, Wo, reg_ln_g, reg_ln_b, sum_ln_g, sum_ln_b,
         reg_fln_g, reg_fln_b, sum_fln_g, sum_fln_b,
         reg_fc1_w, reg_fc1_b, reg_fc2_w, reg_fc2_b,
         sum_fc1_w, sum_fc1_b, sum_fc2_w, sum_fc2_b, interpret=False):
    full = lambda shape: pl.BlockSpec(shape, lambda c: (0,) * len(shape))
    in_specs = [
        pl.BlockSpec((CHUNK_LEN, EMBED_DIM), lambda c: (c, 0)),
        full((SUM_LEN, EMBED_DIM)),
        full((EMBED_DIM, EMBED_DIM)),
        full((EMBED_DIM, EMBED_DIM)),
        full((EMBED_DIM, EMBED_DIM)),
        full((EMBED_DIM, EMBED_DIM)),
        full((1, EMBED_DIM)), full((1, EMBED_DIM)),
        full((1, EMBED_DIM)), full((1, EMBED_DIM)),
        full((1, EMBED_DIM)), full((1, EMBED_DIM)),
        full((1, EMBED_DIM)), full((1, EMBED_DIM)),
        full((EMBED_DIM, FFN_DIM)), full((1, FFN_DIM)),
        full((FFN_DIM, EMBED_DIM)), full((1, EMBED_DIM)),
        full((EMBED_DIM, FFN_DIM)), full((1, FFN_DIM)),
        full((FFN_DIM, EMBED_DIM)), full((1, EMBED_DIM)),
    ]
    out_specs = [
        pl.BlockSpec((CHUNK_LEN, EMBED_DIM), lambda c: (c, 0)),
        full((SUM_LEN, EMBED_DIM)),
    ]
    out_reg, out_sum = pl.pallas_call(
        _body,
        grid=(NUM_CHUNKS,),
        in_specs=in_specs,
        out_specs=out_specs,
        out_shape=[
            jax.ShapeDtypeStruct((REG_LEN, EMBED_DIM), jnp.float32),
            jax.ShapeDtypeStruct((SUM_LEN, EMBED_DIM), jnp.float32),
        ],
        scratch_shapes=[
            pltpu.VMEM((SUM_LEN, EMBED_DIM), jnp.float32),
            pltpu.VMEM((SUM_LEN, EMBED_DIM), jnp.float32),
            pltpu.VMEM((SUM_LEN, EMBED_DIM), jnp.float32),
            pltpu.VMEM((EMBED_DIM, NSUM), jnp.float32),
            pltpu.VMEM((REG_LEN, NSUM), jnp.float32),
            pltpu.VMEM((REG_LEN, EMBED_DIM), jnp.float32),
        ],
        compiler_params=pltpu.CompilerParams(
            vmem_limit_bytes=63 * 1024 * 1024),
        interpret=interpret,
    )(
        reg_x[0], sum_x[0], Wq, Wk, Wv, Wo,
        reg_ln_g[None], reg_ln_b[None], sum_ln_g[None], sum_ln_b[None],
        reg_fln_g[None], reg_fln_b[None], sum_fln_g[None], sum_fln_b[None],
        reg_fc1_w, reg_fc1_b[None], reg_fc2_w, reg_fc2_b[None],
        sum_fc1_w, sum_fc1_b[None], sum_fc2_w, sum_fc2_b[None],
    )
    return jnp.concatenate([out_sum, out_reg], axis=0)[None]


def kernel(reg_x, sum_x, Wq, Wk, Wv, Wo, reg_ln_g, reg_ln_b, sum_ln_g,
           sum_ln_b, reg_fln_g, reg_fln_b, sum_fln_g, sum_fln_b,
           reg_fc1_w, reg_fc1_b, reg_fc2_w, reg_fc2_b,
           sum_fc1_w, sum_fc1_b, sum_fc2_w, sum_fc2_b):
    return _run(reg_x, sum_x, Wq, Wk, Wv, Wo, reg_ln_g, reg_ln_b, sum_ln_g,
                sum_ln_b, reg_fln_g, reg_fln_b, sum_fln_g, sum_fln_b,
                reg_fc1_w, reg_fc1_b, reg_fc2_w, reg_fc2_b,
                sum_fc1_w, sum_fc1_b, sum_fc2_w, sum_fc2_b)


# speedup vs baseline: 1.2466x; 1.0471x over previous
import functools

import jax
import jax.numpy as jnp
from jax.experimental import pallas as pl
from jax.experimental.pallas import tpu as pltpu

EMBED_DIM = 768
FFN_DIM = 3072
NUM_HEADS = 12
HEAD_DIM = EMBED_DIM // NUM_HEADS
CHUNK_LEN = 256
REG_LEN = 2048
NUM_CHUNKS = REG_LEN // CHUNK_LEN
SUM_LEN = NUM_CHUNKS
NSUM = NUM_HEADS * SUM_LEN
SCALE = 1.0 / (HEAD_DIM ** 0.5)


def _ln(x, g, b):
    m = jnp.mean(x, axis=-1, keepdims=True)
    v = jnp.mean((x - m) ** 2, axis=-1, keepdims=True)
    return (x - m) * jax.lax.rsqrt(v + 1e-5) * g + b


def _dot(a, b):
    return jnp.dot(a, b, preferred_element_type=jnp.float32)


def _dot_t(a, b):
    return jax.lax.dot_general(a, b, (((1,), (1,)), ((), ())),
                               preferred_element_type=jnp.float32)


def _dot_tl(a, b):
    return jax.lax.dot_general(a, b, (((0,), (0,)), ((), ())),
                               preferred_element_type=jnp.float32)


def _body(reg_x_ref, sum_x_ref, wq_ref, wk_ref, wv_ref, wo_ref,
          reg_ln_g_ref, reg_ln_b_ref, sum_ln_g_ref, sum_ln_b_ref,
          reg_fln_g_ref, reg_fln_b_ref, sum_fln_g_ref, sum_fln_b_ref,
          rfc1w_ref, rfc1b_ref, rfc2w_ref, rfc2b_ref,
          sfc1w_ref, sfc1b_ref, sfc2w_ref, sfc2b_ref,
          out_ref,
          qs_ref, ks_ref, vs_ref, qbd_ref, ssc_ref, vall_ref):
    c = pl.program_id(0)

    @pl.when(c == 0)
    def _init_summary_qkv():
        hs = _ln(sum_x_ref[...], sum_ln_g_ref[...], sum_ln_b_ref[...])
        q_sum = _dot(hs, wq_ref[...]) * SCALE
        k_sum = _dot(hs, wk_ref[...])
        v_sum = _dot(hs, wv_ref[...])
        qs_ref[...] = q_sum
        ks_ref[...] = k_sum
        vs_ref[...] = v_sum
        qbd_ref[...] = jnp.zeros((EMBED_DIM, NSUM), jnp.float32)
        for hd in range(NUM_HEADS):
            sl = slice(hd * HEAD_DIM, (hd + 1) * HEAD_DIM)
            ssl = slice(hd * SUM_LEN, (hd + 1) * SUM_LEN)
            qbd_ref[sl, ssl] = q_sum[:, sl].T

    x0 = reg_x_ref[...]
    h = _ln(x0, reg_ln_g_ref[...], reg_ln_b_ref[...])
    q = _dot(h, wq_ref[...]) * SCALE
    k = _dot(h, wk_ref[...])
    v = _dot(h, wv_ref[...])
    vall_ref[pl.ds(c * CHUNK_LEN, CHUNK_LEN), :] = v.astype(jnp.bfloat16)

    q_sum = qs_ref[...]
    k_sum = ks_ref[...]
    v_sum = vs_ref[...]

    ssc_ref[pl.ds(c * CHUNK_LEN, CHUNK_LEN), :] = _dot(k, qbd_ref[...])

    row = jax.lax.broadcasted_iota(jnp.int32, (CHUNK_LEN, CHUNK_LEN), 0)
    col = jax.lax.broadcasted_iota(jnp.int32, (CHUNK_LEN, CHUNK_LEN), 1)
    causal_f = (row >= col).astype(jnp.float32)
    col_s = jax.lax.broadcasted_iota(jnp.int32, (CHUNK_LEN, SUM_LEN), 1)
    sum_key_f = (col_s < c).astype(jnp.float32)

    ctxs = []
    for hd in range(NUM_HEADS):
        sl = slice(hd * HEAD_DIM, (hd + 1) * HEAD_DIM)
        qh, kh, vh = q[:, sl], k[:, sl], v[:, sl]
        e_loc = jnp.exp(_dot_t(qh, kh)) * causal_f
        e_sm = jnp.exp(_dot_t(qh, k_sum[:, sl])) * sum_key_f
        l = (jnp.sum(e_loc, axis=-1, keepdims=True)
             + jnp.sum(e_sm, axis=-1, keepdims=True))
        ctxs.append((_dot(e_loc, vh) + _dot(e_sm, v_sum[:, sl])) * (1.0 / l))

    ctx = jnp.concatenate(ctxs, axis=1)
    x = x0 + _dot(ctx, wo_ref[...])
    f = _ln(x, reg_fln_g_ref[...], reg_fln_b_ref[...])
    ffn = jnp.maximum(_dot(f, rfc1w_ref[...]) + rfc1b_ref[...], 0.0)
    out_ref[pl.ds(SUM_LEN + c * CHUNK_LEN, CHUNK_LEN), :] = (
        x + _dot(ffn, rfc2w_ref[...]) + rfc2b_ref[...])

    @pl.when(c == NUM_CHUNKS - 1)
    def _finalize_summary():
        q_sum = qs_ref[...]
        k_sum = ks_ref[...]
        v_sum = vs_ref[...]
        rowr = jax.lax.broadcasted_iota(jnp.int32, (REG_LEN, NSUM), 0)
        colr = jax.lax.broadcasted_iota(jnp.int32, (REG_LEN, NSUM), 1)
        sr_f = ((rowr // CHUNK_LEN) <= (colr & (SUM_LEN - 1))).astype(
            jnp.float32)
        e_sr_all = jnp.exp(ssc_ref[...]) * sr_f
        ctx_sr = _dot_tl(e_sr_all.astype(jnp.bfloat16),
                         vall_ref[...])
        l_sr = jnp.sum(e_sr_all, axis=0, keepdims=True)
        row8 = jax.lax.broadcasted_iota(jnp.int32, (SUM_LEN, SUM_LEN), 0)
        col8 = jax.lax.broadcasted_iota(jnp.int32, (SUM_LEN, SUM_LEN), 1)
        ss_f = (col8 <= row8).astype(jnp.float32)
        ctxs_s = []
        for hd in range(NUM_HEADS):
            sl = slice(hd * HEAD_DIM, (hd + 1) * HEAD_DIM)
            ssl = slice(hd * SUM_LEN, (hd + 1) * SUM_LEN)
            e_ss = jnp.exp(_dot_t(q_sum[:, sl], k_sum[:, sl])) * ss_f
            l = (jnp.sum(e_ss, axis=-1, keepdims=True)
                 + l_sr[:, ssl].T)
            ctxs_s.append((_dot(e_ss, v_sum[:, sl]) + ctx_sr[ssl, sl])
                          * (1.0 / l))
        ctx_s = jnp.concatenate(ctxs_s, axis=1)
        xs = sum_x_ref[...] + _dot(ctx_s, wo_ref[...])
        fs = _ln(xs, sum_fln_g_ref[...], sum_fln_b_ref[...])
        ffn_s = jnp.maximum(_dot(fs, sfc1w_ref[...]) + sfc1b_ref[...], 0.0)
        out_ref[0:SUM_LEN, :] = xs + _dot(ffn_s, sfc2w_ref[...]) + sfc2b_ref[...]


@functools.partial(jax.jit, static_argnames=("interpret",))
def _run(reg_x, sum_x, Wq, Wk, Wv, Wo, reg_ln_g, reg_ln_b, sum_ln_g, sum_ln_b,
         reg_fln_g, reg_fln_b, sum_fln_g, sum_fln_b,
         reg_fc1_w, reg_fc1_b, reg_fc2_w, reg_fc2_b,
         sum_fc1_w, sum_fc1_b, sum_fc2_w, sum_fc2_b, interpret=False):
    full = lambda shape: pl.BlockSpec(shape, lambda c: (0,) * len(shape))
    in_specs = [
        pl.BlockSpec((CHUNK_LEN, EMBED_DIM), lambda c: (c, 0)),
        full((SUM_LEN, EMBED_DIM)),
        full((EMBED_DIM, EMBED_DIM)),
        full((EMBED_DIM, EMBED_DIM)),
        full((EMBED_DIM, EMBED_DIM)),
        full((EMBED_DIM, EMBED_DIM)),
        full((1, EMBED_DIM)), full((1, EMBED_DIM)),
        full((1, EMBED_DIM)), full((1, EMBED_DIM)),
        full((1, EMBED_DIM)), full((1, EMBED_DIM)),
        full((1, EMBED_DIM)), full((1, EMBED_DIM)),
        full((EMBED_DIM, FFN_DIM)), full((1, FFN_DIM)),
        full((FFN_DIM, EMBED_DIM)), full((1, EMBED_DIM)),
        full((EMBED_DIM, FFN_DIM)), full((1, FFN_DIM)),
        full((FFN_DIM, EMBED_DIM)), full((1, EMBED_DIM)),
    ]
    out = pl.pallas_call(
        _body,
        grid=(NUM_CHUNKS,),
        in_specs=in_specs,
        out_specs=full((SUM_LEN + REG_LEN, EMBED_DIM)),
        out_shape=jax.ShapeDtypeStruct((SUM_LEN + REG_LEN, EMBED_DIM),
                                       jnp.float32),
        scratch_shapes=[
            pltpu.VMEM((SUM_LEN, EMBED_DIM), jnp.float32),
            pltpu.VMEM((SUM_LEN, EMBED_DIM), jnp.float32),
            pltpu.VMEM((SUM_LEN, EMBED_DIM), jnp.float32),
            pltpu.VMEM((EMBED_DIM, NSUM), jnp.float32),
            pltpu.VMEM((REG_LEN, NSUM), jnp.float32),
            pltpu.VMEM((REG_LEN, EMBED_DIM), jnp.bfloat16),
        ],
        compiler_params=pltpu.CompilerParams(
            vmem_limit_bytes=63 * 1024 * 1024),
        interpret=interpret,
    )(
        reg_x[0], sum_x[0], Wq, Wk, Wv, Wo,
        reg_ln_g[None], reg_ln_b[None], sum_ln_g[None], sum_ln_b[None],
        reg_fln_g[None], reg_fln_b[None], sum_fln_g[None], sum_fln_b[None],
        reg_fc1_w, reg_fc1_b[None], reg_fc2_w, reg_fc2_b[None],
        sum_fc1_w, sum_fc1_b[None], sum_fc2_w, sum_fc2_b[None],
    )
    return out[None]


def kernel(reg_x, sum_x, Wq, Wk, Wv, Wo, reg_ln_g, reg_ln_b, sum_ln_g,
           sum_ln_b, reg_fln_g, reg_fln_b, sum_fln_g, sum_fln_b,
           reg_fc1_w, reg_fc1_b, reg_fc2_w, reg_fc2_b,
           sum_fc1_w, sum_fc1_b, sum_fc2_w, sum_fc2_b):
    return _run(reg_x, sum_x, Wq, Wk, Wv, Wo, reg_ln_g, reg_ln_b, sum_ln_g,
                sum_ln_b, reg_fln_g, reg_fln_b, sum_fln_g, sum_fln_b,
                reg_fc1_w, reg_fc1_b, reg_fc2_w, reg_fc2_b,
                sum_fc1_w, sum_fc1_b, sum_fc2_w, sum_fc2_b)
